# Initial kernel scaffold; baseline (speedup 1.0000x reference)
import jax
import jax.numpy as jnp
from jax.experimental import pallas as pl


def kernel(fallback_probs, labels, success, perturbation_seeds, lagrange_multiplier):
    raise NotImplementedError("write your pallas kernel here")


# speedup vs baseline: 25.9772x; 25.9772x over previous
import functools

import jax
import jax.numpy as jnp
from jax import lax
from jax.experimental import pallas as pl
from jax.experimental.pallas import tpu as pltpu
from jax.experimental.pallas import tpu_sc as plsc

_NUM_SEEDS = 16
_CVAR_ALPHA = 0.2
_CVAR_EPSILON = 0.3
_COST_SLM = 1.0
_COST_LLM = 50.0
_BRIER_WEIGHT = 1.0

_NC = 2
_NS = 16
_NW = _NC * _NS
_L = 16

_N = 1048576
_PW = _N // _NW
_C = 8192
_NCHUNK = _PW // _C

_mesh = plsc.VectorSubcoreMesh(core_axis_name="c", subcore_axis_name="s")


@functools.partial(
    pl.kernel,
    mesh=_mesh,
    out_type=jax.ShapeDtypeStruct((_NW * 64,), jnp.float32),
    compiler_params=pltpu.CompilerParams(needs_layout_passes=False),
    scratch_types=[
        pltpu.VMEM((_C,), jnp.float32),
        pltpu.VMEM((_C,), jnp.float32),
        pltpu.VMEM((_C,), jnp.float32),
        pltpu.VMEM((_C,), jnp.int32),
        pltpu.VMEM((16,), jnp.float32),
        pltpu.VMEM((16,), jnp.float32),
        pltpu.VMEM((64,), jnp.float32),
    ],
)
def _stage1(fp_hbm, lb_hbm, sc_hbm, sd_hbm, out_hbm,
            fp_v, lb_v, sc_v, sd_v, cnt_v, suc_v, stage_v):
    wid = lax.axis_index("s") * _NC + lax.axis_index("c")
    base = wid * _PW

    zeros16 = jnp.zeros((_L,), jnp.float32)
    ones16 = jnp.ones((_L,), jnp.float32)
    cnt_v[...] = zeros16
    suc_v[...] = zeros16

    acc_fp = zeros16
    acc_sq = zeros16
    for k in range(_NCHUNK):
        off = base + k * _C
        pltpu.sync_copy(fp_hbm.at[pl.ds(off, _C)], fp_v)
        pltpu.sync_copy(lb_hbm.at[pl.ds(off, _C)], lb_v)
        pltpu.sync_copy(sc_hbm.at[pl.ds(off, _C)], sc_v)
        pltpu.sync_copy(sd_hbm.at[pl.ds(off, _C)], sd_v)

        def body(i, carry):
            afp, asq = carry
            j = i * _L
            fp = fp_v[pl.ds(j, _L)]
            lb = lb_v[pl.ds(j, _L)]
            sc = sc_v[pl.ds(j, _L)]
            sd = sd_v[pl.ds(j, _L)]
            plsc.addupdate_scatter(cnt_v, [sd], ones16)
            plsc.addupdate_scatter(suc_v, [sd], sc)
            d = fp - lb
            return afp + fp, asq + d * d

        acc_fp, acc_sq = lax.fori_loop(0, _C // _L, body, (acc_fp, acc_sq))

    stage_v[pl.ds(0, _L)] = cnt_v[...]
    stage_v[pl.ds(16, _L)] = suc_v[...]
    stage_v[pl.ds(32, _L)] = acc_fp
    stage_v[pl.ds(48, _L)] = acc_sq
    pltpu.sync_copy(stage_v, out_hbm.at[pl.ds(wid * 64, 64)])


@functools.partial(
    pl.kernel,
    mesh=_mesh,
    out_type=jax.ShapeDtypeStruct((16,), jnp.float32),
    compiler_params=pltpu.CompilerParams(needs_layout_passes=False),
    scratch_types=[
        pltpu.VMEM((_NW * 64,), jnp.float32),
        pltpu.VMEM((16,), jnp.float32),
    ],
)
def _stage2(parts_hbm, out_hbm, parts_v, out_v):
    wid = lax.axis_index("s") * _NC + lax.axis_index("c")

    @pl.when(wid == 0)
    def _():
        pltpu.sync_copy(parts_hbm, parts_v)

        def body(i, carry):
            c, s, f, q = carry
            j = i * 64
            return (c + parts_v[pl.ds(j, _L)],
                    s + parts_v[pl.ds(j + 16, _L)],
                    f + parts_v[pl.ds(j + 32, _L)],
                    q + parts_v[pl.ds(j + 48, _L)])

        zeros16 = jnp.zeros((_L,), jnp.float32)
        cnt, suc, fps, sqs = lax.fori_loop(
            0, _NW, body, (zeros16, zeros16, zeros16, zeros16))

        idx = lax.iota(jnp.int32, _L)
        seed_fail = (cnt - suc) / jnp.maximum(cnt, 1.0)
        skey, _sval = plsc.sort_key_val(seed_fail, idx, descending=True)
        kk = max(1, int(_NUM_SEEDS * _CVAR_ALPHA))
        rob = jnp.sum(jnp.where(idx < kk, skey, 0.0)) * (1.0 / kk)
        sum_fp = jnp.sum(fps)
        sum_sq = jnp.sum(sqs)

        out_v[...] = jnp.where(
            idx == 0, jnp.full((_L,), sum_fp),
            jnp.where(idx == 1, jnp.full((_L,), sum_sq),
                      jnp.where(idx == 2, jnp.full((_L,), rob), zeros16)))
        pltpu.sync_copy(out_v, out_hbm)


def kernel(fallback_probs, labels, success, perturbation_seeds, lagrange_multiplier):
    n = fallback_probs.shape[0]
    seeds32 = perturbation_seeds.astype(jnp.int32)
    parts = _stage1(fallback_probs, labels, success, seeds32)
    vec = _stage2(parts)

    inv_n = 1.0 / n
    cost_loss = (_COST_LLM - _COST_SLM) * (vec[0] * inv_n) + _COST_SLM
    brier_loss = vec[1] * inv_n
    robustness_loss = vec[2]
    constraint_violation = robustness_loss - _CVAR_EPSILON
    lagrangian_term = lagrange_multiplier * constraint_violation
    total_loss = cost_loss + lagrangian_term + _BRIER_WEIGHT * brier_loss
    dual_loss = -lagrange_multiplier * lax.stop_gradient(constraint_violation)
    return (
        total_loss,
        cost_loss,
        robustness_loss,
        brier_loss,
        lagrangian_term,
        dual_loss,
        lax.stop_gradient(constraint_violation),
    )


# speedup vs baseline: 31.3650x; 1.2074x over previous
import functools

import jax
import jax.numpy as jnp
from jax import lax
from jax.experimental import pallas as pl
from jax.experimental.pallas import tpu as pltpu
from jax.experimental.pallas import tpu_sc as plsc

_NUM_SEEDS = 16
_CVAR_ALPHA = 0.2
_CVAR_EPSILON = 0.3
_COST_SLM = 1.0
_COST_LLM = 50.0
_BRIER_WEIGHT = 1.0

_NC = 2
_NS = 16
_NW = _NC * _NS
_L = 16

_N = 1048576
_PW = _N // _NW
_C = 4096
_NCHUNK = _PW // _C
_U = 4

_mesh = plsc.VectorSubcoreMesh(core_axis_name="c", subcore_axis_name="s")


@functools.partial(
    pl.kernel,
    mesh=_mesh,
    out_type=jax.ShapeDtypeStruct((_NW * 64,), jnp.float32),
    compiler_params=pltpu.CompilerParams(needs_layout_passes=False),
    scratch_types=[
        pltpu.VMEM((_C,), jnp.float32),
        pltpu.VMEM((_C,), jnp.float32),
        pltpu.VMEM((_C,), jnp.float32),
        pltpu.VMEM((_C,), jnp.int32),
        pltpu.VMEM((_C,), jnp.float32),
        pltpu.VMEM((_C,), jnp.float32),
        pltpu.VMEM((_C,), jnp.float32),
        pltpu.VMEM((_C,), jnp.int32),
        pltpu.SemaphoreType.DMA,
        pltpu.SemaphoreType.DMA,
        pltpu.VMEM((16,), jnp.float32),
        pltpu.VMEM((16,), jnp.float32),
        pltpu.VMEM((64,), jnp.float32),
    ],
)
def _stage1(fp_hbm, lb_hbm, sc_hbm, sd_hbm, out_hbm,
            fp0, lb0, sc0, sd0, fp1, lb1, sc1, sd1, sem0, sem1,
            cnt_v, suc_v, stage_v):
    wid = lax.axis_index("s") * _NC + lax.axis_index("c")
    base = wid * _PW

    bufs = ((fp0, lb0, sc0, sd0), (fp1, lb1, sc1, sd1))
    sems = (sem0, sem1)

    def fire(k):
        slot = k % 2
        off = base + k * _C
        b = bufs[slot]
        sem = sems[slot]
        return [
            pltpu.async_copy(fp_hbm.at[pl.ds(off, _C)], b[0], sem),
            pltpu.async_copy(lb_hbm.at[pl.ds(off, _C)], b[1], sem),
            pltpu.async_copy(sc_hbm.at[pl.ds(off, _C)], b[2], sem),
            pltpu.async_copy(sd_hbm.at[pl.ds(off, _C)], b[3], sem),
        ]

    zeros16 = jnp.zeros((_L,), jnp.float32)
    ones16 = jnp.ones((_L,), jnp.float32)
    cnt_v[...] = zeros16
    suc_v[...] = zeros16

    acc_fp = [zeros16] * _U
    acc_sq = [zeros16] * _U
    handles = {0: fire(0)}
    for k in range(_NCHUNK):
        if k + 1 < _NCHUNK:
            handles[k + 1] = fire(k + 1)
        for h in handles.pop(k):
            h.wait()
        fp_v, lb_v, sc_v, sd_v = bufs[k % 2]

        def body(i, carry):
            afp, asq = carry
            afp, asq = list(afp), list(asq)
            for u in range(_U):
                j = i * (_L * _U) + u * _L
                fp = fp_v[pl.ds(j, _L)]
                lb = lb_v[pl.ds(j, _L)]
                sc = sc_v[pl.ds(j, _L)]
                sd = sd_v[pl.ds(j, _L)]
                plsc.addupdate_scatter(cnt_v, [sd], ones16)
                plsc.addupdate_scatter(suc_v, [sd], sc)
                d = fp - lb
                afp[u] = afp[u] + fp
                asq[u] = asq[u] + d * d
            return tuple(afp), tuple(asq)

        acc_fp, acc_sq = lax.fori_loop(
            0, _C // (_L * _U), body, (tuple(acc_fp), tuple(acc_sq)))

    acc_fp = sum(acc_fp[1:], acc_fp[0])
    acc_sq = sum(acc_sq[1:], acc_sq[0])

    stage_v[pl.ds(0, _L)] = cnt_v[...]
    stage_v[pl.ds(16, _L)] = suc_v[...]
    stage_v[pl.ds(32, _L)] = acc_fp
    stage_v[pl.ds(48, _L)] = acc_sq
    pltpu.sync_copy(stage_v, out_hbm.at[pl.ds(wid * 64, 64)])


@functools.partial(
    pl.kernel,
    mesh=_mesh,
    out_type=jax.ShapeDtypeStruct((16,), jnp.float32),
    compiler_params=pltpu.CompilerParams(needs_layout_passes=False),
    scratch_types=[
        pltpu.VMEM((_NW * 64,), jnp.float32),
        pltpu.VMEM((16,), jnp.float32),
    ],
)
def _stage2(parts_hbm, out_hbm, parts_v, out_v):
    wid = lax.axis_index("s") * _NC + lax.axis_index("c")

    @pl.when(wid == 0)
    def _():
        pltpu.sync_copy(parts_hbm, parts_v)

        def body(i, carry):
            c, s, f, q = carry
            j = i * 64
            return (c + parts_v[pl.ds(j, _L)],
                    s + parts_v[pl.ds(j + 16, _L)],
                    f + parts_v[pl.ds(j + 32, _L)],
                    q + parts_v[pl.ds(j + 48, _L)])

        zeros16 = jnp.zeros((_L,), jnp.float32)
        cnt, suc, fps, sqs = lax.fori_loop(
            0, _NW, body, (zeros16, zeros16, zeros16, zeros16))

        idx = lax.iota(jnp.int32, _L)
        seed_fail = (cnt - suc) / jnp.maximum(cnt, 1.0)
        skey, _sval = plsc.sort_key_val(seed_fail, idx, descending=True)
        kk = max(1, int(_NUM_SEEDS * _CVAR_ALPHA))
        rob = jnp.sum(jnp.where(idx < kk, skey, 0.0)) * (1.0 / kk)
        sum_fp = jnp.sum(fps)
        sum_sq = jnp.sum(sqs)

        out_v[...] = jnp.where(
            idx == 0, jnp.full((_L,), sum_fp),
            jnp.where(idx == 1, jnp.full((_L,), sum_sq),
                      jnp.where(idx == 2, jnp.full((_L,), rob), zeros16)))
        pltpu.sync_copy(out_v, out_hbm)


def kernel(fallback_probs, labels, success, perturbation_seeds, lagrange_multiplier):
    n = fallback_probs.shape[0]
    seeds32 = perturbation_seeds.astype(jnp.int32)
    parts = _stage1(fallback_probs, labels, success, seeds32)
    vec = _stage2(parts)

    inv_n = 1.0 / n
    cost_loss = (_COST_LLM - _COST_SLM) * (vec[0] * inv_n) + _COST_SLM
    brier_loss = vec[1] * inv_n
    robustness_loss = vec[2]
    constraint_violation = robustness_loss - _CVAR_EPSILON
    lagrangian_term = lagrange_multiplier * constraint_violation
    total_loss = cost_loss + lagrangian_term + _BRIER_WEIGHT * brier_loss
    dual_loss = -lagrange_multiplier * lax.stop_gradient(constraint_violation)
    return (
        total_loss,
        cost_loss,
        robustness_loss,
        brier_loss,
        lagrangian_term,
        dual_loss,
        lax.stop_gradient(constraint_violation),
    )


# speedup vs baseline: 31.5701x; 1.0065x over previous
import functools

import jax
import jax.numpy as jnp
from jax import lax
from jax.experimental import pallas as pl
from jax.experimental.pallas import tpu as pltpu
from jax.experimental.pallas import tpu_sc as plsc

_NUM_SEEDS = 16
_CVAR_ALPHA = 0.2
_CVAR_EPSILON = 0.3
_COST_SLM = 1.0
_COST_LLM = 50.0
_BRIER_WEIGHT = 1.0

_NC = 2
_NS = 16
_NW = _NC * _NS
_L = 16

_N = 1048576
_PW = _N // _NW
_C = 4096
_NCHUNK = _PW // _C
_U = 4

_mesh = plsc.VectorSubcoreMesh(core_axis_name="c", subcore_axis_name="s")


@functools.partial(
    pl.kernel,
    mesh=_mesh,
    out_type=jax.ShapeDtypeStruct((_NW * 64,), jnp.float32),
    compiler_params=pltpu.CompilerParams(needs_layout_passes=False),
    scratch_types=[
        pltpu.VMEM((_C,), jnp.float32),
        pltpu.VMEM((_C,), jnp.float32),
        pltpu.VMEM((_C,), jnp.float32),
        pltpu.VMEM((_C,), jnp.int32),
        pltpu.VMEM((_C,), jnp.float32),
        pltpu.VMEM((_C,), jnp.float32),
        pltpu.VMEM((_C,), jnp.float32),
        pltpu.VMEM((_C,), jnp.int32),
        pltpu.SemaphoreType.DMA,
        pltpu.SemaphoreType.DMA,
        pltpu.VMEM((_U, 16), jnp.float32),
        pltpu.VMEM((_U, 16), jnp.float32),
        pltpu.VMEM((64,), jnp.float32),
    ],
)
def _stage1(fp_hbm, lb_hbm, sc_hbm, sd_hbm, out_hbm,
            fp0, lb0, sc0, sd0, fp1, lb1, sc1, sd1, sem0, sem1,
            cnt_v, suc_v, stage_v):
    wid = lax.axis_index("s") * _NC + lax.axis_index("c")
    base = wid * _PW

    bufs = ((fp0, lb0, sc0, sd0), (fp1, lb1, sc1, sd1))
    sems = (sem0, sem1)

    def fire(k):
        slot = k % 2
        off = base + k * _C
        b = bufs[slot]
        sem = sems[slot]
        return [
            pltpu.async_copy(fp_hbm.at[pl.ds(off, _C)], b[0], sem),
            pltpu.async_copy(lb_hbm.at[pl.ds(off, _C)], b[1], sem),
            pltpu.async_copy(sc_hbm.at[pl.ds(off, _C)], b[2], sem),
            pltpu.async_copy(sd_hbm.at[pl.ds(off, _C)], b[3], sem),
        ]

    zeros16 = jnp.zeros((_L,), jnp.float32)
    ones16 = jnp.ones((_L,), jnp.float32)
    for u in range(_U):
        cnt_v[u, :] = zeros16
        suc_v[u, :] = zeros16

    acc_fp = [zeros16] * _U
    acc_sq = [zeros16] * _U
    handles = {0: fire(0)}
    for k in range(_NCHUNK):
        if k + 1 < _NCHUNK:
            handles[k + 1] = fire(k + 1)
        for h in handles.pop(k):
            h.wait()
        fp_v, lb_v, sc_v, sd_v = bufs[k % 2]

        def body(i, carry):
            afp, asq = carry
            afp, asq = list(afp), list(asq)
            for u in range(_U):
                j = i * (_L * _U) + u * _L
                fp = fp_v[pl.ds(j, _L)]
                lb = lb_v[pl.ds(j, _L)]
                sc = sc_v[pl.ds(j, _L)]
                sd = sd_v[pl.ds(j, _L)]
                plsc.addupdate_scatter(cnt_v.at[u], [sd], ones16)
                plsc.addupdate_scatter(suc_v.at[u], [sd], sc)
                d = fp - lb
                afp[u] = afp[u] + fp
                asq[u] = asq[u] + d * d
            return tuple(afp), tuple(asq)

        acc_fp, acc_sq = lax.fori_loop(
            0, _C // (_L * _U), body, (tuple(acc_fp), tuple(acc_sq)))

    acc_fp = sum(acc_fp[1:], acc_fp[0])
    acc_sq = sum(acc_sq[1:], acc_sq[0])
    cnt = cnt_v[0, :]
    suc = suc_v[0, :]
    for u in range(1, _U):
        cnt = cnt + cnt_v[u, :]
        suc = suc + suc_v[u, :]

    stage_v[pl.ds(0, _L)] = cnt
    stage_v[pl.ds(16, _L)] = suc
    stage_v[pl.ds(32, _L)] = acc_fp
    stage_v[pl.ds(48, _L)] = acc_sq
    pltpu.sync_copy(stage_v, out_hbm.at[pl.ds(wid * 64, 64)])


@functools.partial(
    pl.kernel,
    mesh=_mesh,
    out_type=jax.ShapeDtypeStruct((16,), jnp.float32),
    compiler_params=pltpu.CompilerParams(needs_layout_passes=False),
    scratch_types=[
        pltpu.VMEM((_NW * 64,), jnp.float32),
        pltpu.VMEM((16,), jnp.float32),
    ],
)
def _stage2(parts_hbm, out_hbm, parts_v, out_v):
    wid = lax.axis_index("s") * _NC + lax.axis_index("c")

    @pl.when(wid == 0)
    def _():
        pltpu.sync_copy(parts_hbm, parts_v)

        def body(i, carry):
            c, s, f, q = carry
            j = i * 64
            return (c + parts_v[pl.ds(j, _L)],
                    s + parts_v[pl.ds(j + 16, _L)],
                    f + parts_v[pl.ds(j + 32, _L)],
                    q + parts_v[pl.ds(j + 48, _L)])

        zeros16 = jnp.zeros((_L,), jnp.float32)
        cnt, suc, fps, sqs = lax.fori_loop(
            0, _NW, body, (zeros16, zeros16, zeros16, zeros16))

        idx = lax.iota(jnp.int32, _L)
        seed_fail = (cnt - suc) / jnp.maximum(cnt, 1.0)
        skey, _sval = plsc.sort_key_val(seed_fail, idx, descending=True)
        kk = max(1, int(_NUM_SEEDS * _CVAR_ALPHA))
        rob = jnp.sum(jnp.where(idx < kk, skey, 0.0)) * (1.0 / kk)
        sum_fp = jnp.sum(fps)
        sum_sq = jnp.sum(sqs)

        out_v[...] = jnp.where(
            idx == 0, jnp.full((_L,), sum_fp),
            jnp.where(idx == 1, jnp.full((_L,), sum_sq),
                      jnp.where(idx == 2, jnp.full((_L,), rob), zeros16)))
        pltpu.sync_copy(out_v, out_hbm)


def kernel(fallback_probs, labels, success, perturbation_seeds, lagrange_multiplier):
    n = fallback_probs.shape[0]
    seeds32 = perturbation_seeds.astype(jnp.int32)
    parts = _stage1(fallback_probs, labels, success, seeds32)
    vec = _stage2(parts)

    inv_n = 1.0 / n
    cost_loss = (_COST_LLM - _COST_SLM) * (vec[0] * inv_n) + _COST_SLM
    brier_loss = vec[1] * inv_n
    robustness_loss = vec[2]
    constraint_violation = robustness_loss - _CVAR_EPSILON
    lagrangian_term = lagrange_multiplier * constraint_violation
    total_loss = cost_loss + lagrangian_term + _BRIER_WEIGHT * brier_loss
    dual_loss = -lagrange_multiplier * lax.stop_gradient(constraint_violation)
    return (
        total_loss,
        cost_loss,
        robustness_loss,
        brier_loss,
        lagrangian_term,
        dual_loss,
        lax.stop_gradient(constraint_violation),
    )
